# Initial kernel scaffold; baseline (speedup 1.0000x reference)
import jax
import jax.numpy as jnp
from jax.experimental import pallas as pl


def kernel(input_tensors, batch, degree, alpha, gamma, beta):
    raise NotImplementedError("write your pallas kernel here")


# speedup vs baseline: 6.7273x; 6.7273x over previous
import functools

import jax
import jax.numpy as jnp
from jax import lax
from jax.experimental import pallas as pl
from jax.experimental.pallas import tpu as pltpu
from jax.experimental.pallas import tpu_sc as plsc

N = 320000
CDIM = 128
G = 1024
EPS = 0.001

NC = 2
NS = 16
NW = NC * NS
L = 16
NJ = CDIM // L
RPW = N // NW
CH = 400
NCH = RPW // CH
GPC = CH // L
GPAD = G + L

_PARAMS = pltpu.CompilerParams(needs_layout_passes=False)


def _mesh():
    return plsc.VectorSubcoreMesh(
        core_axis_name="c", subcore_axis_name="s", num_cores=NC, num_subcores=NS
    )


def _lane_splat(v, lane):
    idx = jnp.full((L, 1), lane, jnp.int32)
    dnums = lax.GatherDimensionNumbers(
        offset_dims=(), collapsed_slice_dims=(0,), start_index_map=(0,)
    )
    return lax.gather(
        v, idx, dnums, (1,), mode=lax.GatherScatterMode.PROMISE_IN_BOUNDS
    )


def _stats_body(x_hbm, b_hbm, s_out, q_out, c_out,
                vbuf, vids, sqbuf, f_s, f_q, f_c, ones_b, zbuf, zcbuf,
                shared_s, shared_q, shared_c):
    cid = lax.axis_index("c")
    sid = lax.axis_index("s")
    wid = cid * NS + sid
    lanes = lax.iota(jnp.int32, L)
    zero16 = jnp.zeros((L,), jnp.float32)

    for r in range(L):
        ones_b[r, pl.ds(0, L)] = jnp.full((L,), 1.0, jnp.float32)
        zcbuf[r, pl.ds(0, L)] = zero16
        for j in range(NJ):
            zbuf[r, pl.ds(j * L, L)] = zero16

    @pl.when(sid == 0)
    def _():
        def zero_chunk(k, c):
            pltpu.sync_copy(zbuf, shared_s.at[pl.ds(k * L, L)])
            pltpu.sync_copy(zbuf, shared_q.at[pl.ds(k * L, L)])
            pltpu.sync_copy(zcbuf, shared_c.at[pl.ds(k * L, L)])
            return c

        lax.fori_loop(0, GPAD // L, zero_chunk, 0)

    plsc.subcore_barrier()

    def flush(g, acc_c, acc_s, acc_q):
        for j in range(NJ):
            f_s[0, pl.ds(j * L, L)] = acc_s[j]
            f_q[0, pl.ds(j * L, L)] = acc_q[j]
        f_c[0, pl.ds(0, L)] = acc_c
        fidx = jnp.where((lanes == 0) & (g >= 0), g, jnp.int32(G))
        pltpu.sync_copy(f_s, shared_s.at[fidx], add=True)
        pltpu.sync_copy(f_q, shared_q.at[fidx], add=True)
        pltpu.sync_copy(f_c, shared_c.at[fidx], add=True)

    def chunk_body(ch, carry):
        base = wid * RPW + ch * CH
        pltpu.sync_copy(x_hbm.at[pl.ds(base, CH)], vbuf)
        pltpu.sync_copy(b_hbm.at[pl.ds(base, CH)], vids)

        def group_body(gi, carry):
            g = carry[0]
            acc_c = carry[1]
            acc_s = carry[2:2 + NJ]
            acc_q = carry[2 + NJ:2 + 2 * NJ]
            idvec = vids[pl.ds(gi * L, L)]
            nb = jnp.sum(jnp.where(idvec != g, 1.0, 0.0))

            def fast(op):
                g, acc_c, acc_s, acc_q = op
                acc_s = list(acc_s)
                acc_q = list(acc_q)
                for r in range(L):
                    for j in range(NJ):
                        v = vbuf[gi * L + r, pl.ds(j * L, L)]
                        acc_s[j] = acc_s[j] + v
                        acc_q[j] = acc_q[j] + v * v
                return g, acc_c + jnp.float32(L), tuple(acc_s), tuple(acc_q)

            def slow(op):
                g, acc_c, acc_s, acc_q = op
                flush(g, acc_c, acc_s, acc_q)
                for r in range(L):
                    for j in range(NJ):
                        v = vbuf[gi * L + r, pl.ds(j * L, L)]
                        sqbuf[r, pl.ds(j * L, L)] = v * v
                pltpu.sync_copy(
                    vbuf.at[pl.ds(gi * L, L)], shared_s.at[idvec], add=True
                )
                pltpu.sync_copy(sqbuf, shared_q.at[idvec], add=True)
                pltpu.sync_copy(ones_b, shared_c.at[idvec], add=True)
                g = _lane_splat(idvec, L - 1)
                return (g, jnp.zeros((L,), jnp.float32),
                        tuple(zero16 for _ in range(NJ)),
                        tuple(zero16 for _ in range(NJ)))

            g, acc_c, acc_s, acc_q = lax.cond(
                nb == 0.0, fast, slow, (g, acc_c, acc_s, acc_q)
            )
            return (g, acc_c, *acc_s, *acc_q)

        return lax.fori_loop(0, GPC, group_body, carry)

    init = (jnp.full((L,), -1, jnp.int32), jnp.zeros((L,), jnp.float32))
    init = init + tuple(zero16 for _ in range(2 * NJ))
    carry = lax.fori_loop(0, NCH, chunk_body, init)
    flush(carry[0], carry[1], carry[2:2 + NJ], carry[2 + NJ:2 + 2 * NJ])

    plsc.subcore_barrier()

    @pl.when(sid == 0)
    def _():
        pltpu.sync_copy(shared_s.at[pl.ds(0, G)], s_out.at[pl.ds(cid * G, G)])
        pltpu.sync_copy(shared_q.at[pl.ds(0, G)], q_out.at[pl.ds(cid * G, G)])
        pltpu.sync_copy(shared_c.at[pl.ds(0, G)], c_out.at[pl.ds(cid * G, G)])


def _stats(x, b32):
    f = functools.partial(
        pl.kernel,
        out_type=(
            jax.ShapeDtypeStruct((NC * G, CDIM), jnp.float32),
            jax.ShapeDtypeStruct((NC * G, CDIM), jnp.float32),
            jax.ShapeDtypeStruct((NC * G, L), jnp.float32),
        ),
        mesh=_mesh(),
        compiler_params=_PARAMS,
        scratch_types=[
            pltpu.VMEM((CH, CDIM), jnp.float32),
            pltpu.VMEM((CH,), jnp.int32),
            pltpu.VMEM((L, CDIM), jnp.float32),
            pltpu.VMEM((L, CDIM), jnp.float32),
            pltpu.VMEM((L, CDIM), jnp.float32),
            pltpu.VMEM((L, L), jnp.float32),
            pltpu.VMEM((L, L), jnp.float32),
            pltpu.VMEM((L, CDIM), jnp.float32),
            pltpu.VMEM((L, L), jnp.float32),
            pltpu.VMEM_SHARED((GPAD, CDIM), jnp.float32),
            pltpu.VMEM_SHARED((GPAD, CDIM), jnp.float32),
            pltpu.VMEM_SHARED((GPAD, L), jnp.float32),
        ],
    )
    return f(_stats_body)(x, b32)


def _finalize_body(s_ref, q_ref, c_ref, d_ref, a_ref, g_ref, b_ref,
                   scale_ref, shift_ref):
    s = s_ref[0] + s_ref[1]
    q = q_ref[0] + q_ref[1]
    n = c_ref[0, :, 0:1] + c_ref[1, :, 0:1]
    d = d_ref[...]
    m2 = (s / d) * a_ref[...]
    varsum = q - (2.0 * m2) * s + n * (m2 * m2)
    inv = lax.rsqrt(varsum / d + EPS)
    sc = inv * g_ref[...]
    scale_ref[...] = sc
    shift_ref[...] = b_ref[...] - m2 * sc


def _finalize(s2, q2, c2, degree, alpha, gamma, beta):
    return pl.pallas_call(
        _finalize_body,
        out_shape=(
            jax.ShapeDtypeStruct((G, CDIM), jnp.float32),
            jax.ShapeDtypeStruct((G, CDIM), jnp.float32),
        ),
    )(
        s2.reshape(NC, G, CDIM),
        q2.reshape(NC, G, CDIM),
        c2.reshape(NC, G, L),
        degree.reshape(G, 1),
        alpha.reshape(1, CDIM),
        gamma.reshape(1, CDIM),
        beta.reshape(1, CDIM),
    )


def _apply_body(x_hbm, b_hbm, sc_hbm, sh_hbm, out_hbm,
                vbuf, vids, scbuf, shbuf, shared_sc, shared_sh):
    cid = lax.axis_index("c")
    sid = lax.axis_index("s")
    wid = cid * NS + sid

    @pl.when(sid == 0)
    def _():
        pltpu.sync_copy(sc_hbm, shared_sc)
        pltpu.sync_copy(sh_hbm, shared_sh)

    plsc.subcore_barrier()

    def chunk_body(ch, carry):
        base = wid * RPW + ch * CH
        pltpu.sync_copy(x_hbm.at[pl.ds(base, CH)], vbuf)
        pltpu.sync_copy(b_hbm.at[pl.ds(base, CH)], vids)

        def group_body(gi, carry):
            g = carry[0]
            sc = carry[1:1 + NJ]
            sh = carry[1 + NJ:1 + 2 * NJ]
            idvec = vids[pl.ds(gi * L, L)]
            nb = jnp.sum(jnp.where(idvec != g, 1.0, 0.0))

            def fast(op):
                g, sc, sh = op
                for r in range(L):
                    for j in range(NJ):
                        v = vbuf[gi * L + r, pl.ds(j * L, L)]
                        vbuf[gi * L + r, pl.ds(j * L, L)] = v * sc[j] + sh[j]
                return g, sc, sh

            def slow(op):
                g, sc, sh = op
                pltpu.sync_copy(shared_sc.at[idvec], scbuf)
                pltpu.sync_copy(shared_sh.at[idvec], shbuf)
                for r in range(L):
                    for j in range(NJ):
                        v = vbuf[gi * L + r, pl.ds(j * L, L)]
                        vbuf[gi * L + r, pl.ds(j * L, L)] = (
                            v * scbuf[r, pl.ds(j * L, L)]
                            + shbuf[r, pl.ds(j * L, L)]
                        )
                g = _lane_splat(idvec, L - 1)
                sc = tuple(scbuf[L - 1, pl.ds(j * L, L)] for j in range(NJ))
                sh = tuple(shbuf[L - 1, pl.ds(j * L, L)] for j in range(NJ))
                return g, sc, sh

            g, sc, sh = lax.cond(nb == 0.0, fast, slow, (g, sc, sh))
            return (g, *sc, *sh)

        carry = lax.fori_loop(0, GPC, group_body, carry)
        pltpu.sync_copy(vbuf, out_hbm.at[pl.ds(base, CH)])
        return carry

    init = (jnp.full((L,), -1, jnp.int32),)
    init = init + tuple(jnp.zeros((L,), jnp.float32) for _ in range(2 * NJ))
    lax.fori_loop(0, NCH, chunk_body, init)


def _apply(x, b32, scale, shift):
    f = functools.partial(
        pl.kernel,
        out_type=jax.ShapeDtypeStruct((N, CDIM), jnp.float32),
        mesh=_mesh(),
        compiler_params=_PARAMS,
        scratch_types=[
            pltpu.VMEM((CH, CDIM), jnp.float32),
            pltpu.VMEM((CH,), jnp.int32),
            pltpu.VMEM((L, CDIM), jnp.float32),
            pltpu.VMEM((L, CDIM), jnp.float32),
            pltpu.VMEM_SHARED((G, CDIM), jnp.float32),
            pltpu.VMEM_SHARED((G, CDIM), jnp.float32),
        ],
    )
    return f(_apply_body)(x, b32, scale, shift)


def kernel(input_tensors, batch, degree, alpha, gamma, beta):
    x = input_tensors
    b32 = batch.astype(jnp.int32)
    s2, q2, c2 = _stats(x, b32)
    scale, shift = _finalize(s2, q2, c2, degree, alpha, gamma, beta)
    return _apply(x, b32, scale, shift)


# speedup vs baseline: 9.0043x; 1.3385x over previous
import functools

import jax
import jax.numpy as jnp
from jax import lax
from jax.experimental import pallas as pl
from jax.experimental.pallas import tpu as pltpu
from jax.experimental.pallas import tpu_sc as plsc

N = 320000
CDIM = 128
G = 1024
EPS = 0.001

NC = 2
NS = 16
NW = NC * NS
L = 16
NJ = CDIM // L
RPW = N // NW
GPAD = G + L

CH_S = 80
NCH_S = RPW // CH_S
GPC_S = CH_S // L

CH_A = 80
NCH_A = RPW // CH_A
GPC_A = CH_A // L

_PARAMS = pltpu.CompilerParams(needs_layout_passes=False)


def _mesh():
    return plsc.VectorSubcoreMesh(
        core_axis_name="c", subcore_axis_name="s", num_cores=NC, num_subcores=NS
    )


def _lane_splat(v, lane):
    idx = jnp.full((L, 1), lane, jnp.int32)
    dnums = lax.GatherDimensionNumbers(
        offset_dims=(), collapsed_slice_dims=(0,), start_index_map=(0,)
    )
    return lax.gather(
        v, idx, dnums, (1,), mode=lax.GatherScatterMode.PROMISE_IN_BOUNDS
    )


def _stats_body(x_hbm, b_hbm, s_out, q_out, c_out,
                vbuf0, vbuf1, ids, sqbuf, f_s, f_q, f_c, ones_b, zbuf, zcbuf,
                shared_s, shared_q, shared_c, isem0, isem1):
    cid = lax.axis_index("c")
    sid = lax.axis_index("s")
    wid = cid * NS + sid
    wbase = wid * RPW
    lanes = lax.iota(jnp.int32, L)
    zero16 = jnp.zeros((L,), jnp.float32)

    for r in range(L):
        ones_b[r, pl.ds(0, L)] = jnp.full((L,), 1.0, jnp.float32)
        zcbuf[r, pl.ds(0, L)] = zero16
        for j in range(NJ):
            zbuf[r, pl.ds(j * L, L)] = zero16

    @pl.when(sid == 0)
    def _():
        def zero_chunk(k, c):
            pltpu.sync_copy(zbuf, shared_s.at[pl.ds(k * L, L)])
            pltpu.sync_copy(zbuf, shared_q.at[pl.ds(k * L, L)])
            pltpu.sync_copy(zcbuf, shared_c.at[pl.ds(k * L, L)])
            return c

        lax.fori_loop(0, GPAD // L, zero_chunk, 0)

    plsc.subcore_barrier()

    pltpu.sync_copy(b_hbm.at[pl.ds(wbase, RPW)], ids)
    vbufs = (vbuf0, vbuf1)
    isems = (isem0, isem1)
    pltpu.async_copy(x_hbm.at[pl.ds(wbase, CH_S)], vbuf0, isem0)
    pltpu.async_copy(x_hbm.at[pl.ds(wbase + CH_S, CH_S)], vbuf1, isem1)

    def flush(g, acc_c, acc_s, acc_q):
        for j in range(NJ):
            f_s[0, pl.ds(j * L, L)] = acc_s[j]
            f_q[0, pl.ds(j * L, L)] = acc_q[j]
        f_c[0, pl.ds(0, L)] = acc_c
        fidx = jnp.where((lanes == 0) & (g >= 0), g, jnp.int32(G))
        pltpu.sync_copy(f_s, shared_s.at[fidx], add=True)
        pltpu.sync_copy(f_q, shared_q.at[fidx], add=True)
        pltpu.sync_copy(f_c, shared_c.at[fidx], add=True)

    def process(ch, vbuf, carry):
        def group_body(gi, carry):
            g = carry[0]
            acc_c = carry[1]
            acc_s = carry[2:2 + NJ]
            acc_q = carry[2 + NJ:2 + 2 * NJ]
            idvec = ids[pl.ds(ch * CH_S + gi * L, L)]
            nb = jnp.sum(jnp.where(idvec != g, 1.0, 0.0))

            def fast(op):
                g, acc_c, acc_s, acc_q = op
                acc_s = list(acc_s)
                acc_q = list(acc_q)
                for r in range(L):
                    for j in range(NJ):
                        v = vbuf[gi * L + r, pl.ds(j * L, L)]
                        acc_s[j] = acc_s[j] + v
                        acc_q[j] = acc_q[j] + v * v
                return g, acc_c + jnp.float32(L), tuple(acc_s), tuple(acc_q)

            def slow(op):
                g, acc_c, acc_s, acc_q = op
                flush(g, acc_c, acc_s, acc_q)

                def sqrow(r, c):
                    for j in range(NJ):
                        v = vbuf[gi * L + r, pl.ds(j * L, L)]
                        sqbuf[r, pl.ds(j * L, L)] = v * v
                    return c

                lax.fori_loop(0, L, sqrow, 0)
                pltpu.sync_copy(
                    vbuf.at[pl.ds(gi * L, L)], shared_s.at[idvec], add=True
                )
                pltpu.sync_copy(sqbuf, shared_q.at[idvec], add=True)
                pltpu.sync_copy(ones_b, shared_c.at[idvec], add=True)
                g = _lane_splat(idvec, L - 1)
                return (g, jnp.zeros((L,), jnp.float32),
                        tuple(zero16 for _ in range(NJ)),
                        tuple(zero16 for _ in range(NJ)))

            g, acc_c, acc_s, acc_q = lax.cond(
                nb == 0.0, fast, slow, (g, acc_c, acc_s, acc_q)
            )
            return (g, acc_c, *acc_s, *acc_q)

        return lax.fori_loop(0, GPC_S, group_body, carry)

    def body(i, carry):
        for b in range(2):
            ch = i * 2 + b
            pltpu.make_async_copy(
                x_hbm.at[pl.ds(0, CH_S)], vbufs[b], isems[b]
            ).wait()
            carry = process(ch, vbufs[b], carry)
            nxt = ch + 2

            @pl.when(nxt < NCH_S)
            def _():
                pltpu.async_copy(
                    x_hbm.at[pl.ds(wbase + nxt * CH_S, CH_S)],
                    vbufs[b], isems[b],
                )
        return carry

    init = (jnp.full((L,), -1, jnp.int32), jnp.zeros((L,), jnp.float32))
    init = init + tuple(zero16 for _ in range(2 * NJ))
    carry = lax.fori_loop(0, (NCH_S - 1) // 2, body, init)
    pltpu.make_async_copy(x_hbm.at[pl.ds(0, CH_S)], vbuf0, isem0).wait()
    carry = process(NCH_S - 1, vbuf0, carry)
    flush(carry[0], carry[1], carry[2:2 + NJ], carry[2 + NJ:2 + 2 * NJ])

    plsc.subcore_barrier()

    @pl.when(sid == 0)
    def _():
        pltpu.sync_copy(shared_s.at[pl.ds(0, G)], s_out.at[pl.ds(cid * G, G)])
        pltpu.sync_copy(shared_q.at[pl.ds(0, G)], q_out.at[pl.ds(cid * G, G)])
        pltpu.sync_copy(shared_c.at[pl.ds(0, G)], c_out.at[pl.ds(cid * G, G)])


def _stats(x, b32):
    f = functools.partial(
        pl.kernel,
        out_type=(
            jax.ShapeDtypeStruct((NC * G, CDIM), jnp.float32),
            jax.ShapeDtypeStruct((NC * G, CDIM), jnp.float32),
            jax.ShapeDtypeStruct((NC * G, L), jnp.float32),
        ),
        mesh=_mesh(),
        compiler_params=_PARAMS,
        scratch_types=[
            pltpu.VMEM((CH_S, CDIM), jnp.float32),
            pltpu.VMEM((CH_S, CDIM), jnp.float32),
            pltpu.VMEM((RPW,), jnp.int32),
            pltpu.VMEM((L, CDIM), jnp.float32),
            pltpu.VMEM((L, CDIM), jnp.float32),
            pltpu.VMEM((L, CDIM), jnp.float32),
            pltpu.VMEM((L, L), jnp.float32),
            pltpu.VMEM((L, L), jnp.float32),
            pltpu.VMEM((L, CDIM), jnp.float32),
            pltpu.VMEM((L, L), jnp.float32),
            pltpu.VMEM_SHARED((GPAD, CDIM), jnp.float32),
            pltpu.VMEM_SHARED((GPAD, CDIM), jnp.float32),
            pltpu.VMEM_SHARED((GPAD, L), jnp.float32),
            pltpu.SemaphoreType.DMA,
            pltpu.SemaphoreType.DMA,
        ],
    )
    return f(_stats_body)(x, b32)


def _finalize_body(s_ref, q_ref, c_ref, d_ref, a_ref, g_ref, b_ref,
                   scale_ref, shift_ref):
    s = s_ref[0] + s_ref[1]
    q = q_ref[0] + q_ref[1]
    n = c_ref[0, :, 0:1] + c_ref[1, :, 0:1]
    d = d_ref[...]
    m2 = (s / d) * a_ref[...]
    varsum = q - (2.0 * m2) * s + n * (m2 * m2)
    inv = lax.rsqrt(varsum / d + EPS)
    sc = inv * g_ref[...]
    scale_ref[...] = sc
    shift_ref[...] = b_ref[...] - m2 * sc


def _finalize(s2, q2, c2, degree, alpha, gamma, beta):
    return pl.pallas_call(
        _finalize_body,
        out_shape=(
            jax.ShapeDtypeStruct((G, CDIM), jnp.float32),
            jax.ShapeDtypeStruct((G, CDIM), jnp.float32),
        ),
    )(
        s2.reshape(NC, G, CDIM),
        q2.reshape(NC, G, CDIM),
        c2.reshape(NC, G, L),
        degree.reshape(G, 1),
        alpha.reshape(1, CDIM),
        gamma.reshape(1, CDIM),
        beta.reshape(1, CDIM),
    )


def _apply_body(x_hbm, b_hbm, sc_hbm, sh_hbm, out_hbm,
                ab0, ab1, ab2, ab3, ids, scbuf, shbuf, shared_sc, shared_sh,
                is0, is1, is2, is3, os0, os1, os2, os3):
    cid = lax.axis_index("c")
    sid = lax.axis_index("s")
    wid = cid * NS + sid
    wbase = wid * RPW

    @pl.when(sid == 0)
    def _():
        pltpu.sync_copy(sc_hbm, shared_sc)
        pltpu.sync_copy(sh_hbm, shared_sh)

    plsc.subcore_barrier()

    pltpu.sync_copy(b_hbm.at[pl.ds(wbase, RPW)], ids)
    bufs = (ab0, ab1, ab2, ab3)
    isems = (is0, is1, is2, is3)
    osems = (os0, os1, os2, os3)
    pltpu.async_copy(x_hbm.at[pl.ds(wbase, CH_A)], ab0, is0)
    pltpu.async_copy(x_hbm.at[pl.ds(wbase + CH_A, CH_A)], ab1, is1)

    def process(ch, vbuf, carry):
        def group_body(gi, carry):
            g = carry[0]
            sc = carry[1:1 + NJ]
            sh = carry[1 + NJ:1 + 2 * NJ]
            idvec = ids[pl.ds(ch * CH_A + gi * L, L)]
            nb = jnp.sum(jnp.where(idvec != g, 1.0, 0.0))

            def fast(op):
                g, sc, sh = op
                for r in range(L):
                    for j in range(NJ):
                        v = vbuf[gi * L + r, pl.ds(j * L, L)]
                        vbuf[gi * L + r, pl.ds(j * L, L)] = v * sc[j] + sh[j]
                return g, sc, sh

            def slow(op):
                g, sc, sh = op
                pltpu.sync_copy(shared_sc.at[idvec], scbuf)
                pltpu.sync_copy(shared_sh.at[idvec], shbuf)

                def row(r, c):
                    for j in range(NJ):
                        v = vbuf[gi * L + r, pl.ds(j * L, L)]
                        vbuf[gi * L + r, pl.ds(j * L, L)] = (
                            v * scbuf[r, pl.ds(j * L, L)]
                            + shbuf[r, pl.ds(j * L, L)]
                        )
                    return c

                lax.fori_loop(0, L, row, 0)
                g = _lane_splat(idvec, L - 1)
                sc = tuple(scbuf[L - 1, pl.ds(j * L, L)] for j in range(NJ))
                sh = tuple(shbuf[L - 1, pl.ds(j * L, L)] for j in range(NJ))
                return g, sc, sh

            g, sc, sh = lax.cond(nb == 0.0, fast, slow, (g, sc, sh))
            return (g, *sc, *sh)

        return lax.fori_loop(0, GPC_A, group_body, carry)

    def body(i, carry):
        for b in range(4):
            ch = i * 4 + b
            pltpu.make_async_copy(
                x_hbm.at[pl.ds(0, CH_A)], bufs[b], isems[b]
            ).wait()
            carry = process(ch, bufs[b], carry)
            pltpu.async_copy(
                bufs[b], out_hbm.at[pl.ds(wbase + ch * CH_A, CH_A)], osems[b]
            )
            tgt = ch + 2
            tb = (b + 2) % 4

            @pl.when(tgt < NCH_A)
            def _():
                @pl.when(tgt >= 4)
                def _():
                    pltpu.make_async_copy(
                        bufs[tb], out_hbm.at[pl.ds(0, CH_A)], osems[tb]
                    ).wait()

                pltpu.async_copy(
                    x_hbm.at[pl.ds(wbase + tgt * CH_A, CH_A)],
                    bufs[tb], isems[tb],
                )
        return carry

    init = (jnp.full((L,), -1, jnp.int32),)
    init = init + tuple(jnp.zeros((L,), jnp.float32) for _ in range(2 * NJ))
    carry = lax.fori_loop(0, (NCH_A - 1) // 4, body, init)
    last = NCH_A - 1
    pltpu.make_async_copy(x_hbm.at[pl.ds(0, CH_A)], bufs[0], isems[0]).wait()
    carry = process(last, bufs[0], carry)
    pltpu.async_copy(
        bufs[0], out_hbm.at[pl.ds(wbase + last * CH_A, CH_A)], osems[0]
    )
    for b in range(4):
        pltpu.make_async_copy(
            bufs[b], out_hbm.at[pl.ds(0, CH_A)], osems[b]
        ).wait()


def _apply(x, b32, scale, shift):
    f = functools.partial(
        pl.kernel,
        out_type=jax.ShapeDtypeStruct((N, CDIM), jnp.float32),
        mesh=_mesh(),
        compiler_params=_PARAMS,
        scratch_types=[
            pltpu.VMEM((CH_A, CDIM), jnp.float32),
            pltpu.VMEM((CH_A, CDIM), jnp.float32),
            pltpu.VMEM((CH_A, CDIM), jnp.float32),
            pltpu.VMEM((CH_A, CDIM), jnp.float32),
            pltpu.VMEM((RPW,), jnp.int32),
            pltpu.VMEM((L, CDIM), jnp.float32),
            pltpu.VMEM((L, CDIM), jnp.float32),
            pltpu.VMEM_SHARED((G, CDIM), jnp.float32),
            pltpu.VMEM_SHARED((G, CDIM), jnp.float32),
            pltpu.SemaphoreType.DMA,
            pltpu.SemaphoreType.DMA,
            pltpu.SemaphoreType.DMA,
            pltpu.SemaphoreType.DMA,
            pltpu.SemaphoreType.DMA,
            pltpu.SemaphoreType.DMA,
            pltpu.SemaphoreType.DMA,
            pltpu.SemaphoreType.DMA,
        ],
    )
    return f(_apply_body)(x, b32, scale, shift)


def kernel(input_tensors, batch, degree, alpha, gamma, beta):
    x = input_tensors
    b32 = batch.astype(jnp.int32)
    s2, q2, c2 = _stats(x, b32)
    scale, shift = _finalize(s2, q2, c2, degree, alpha, gamma, beta)
    return _apply(x, b32, scale, shift)
